# Initial kernel scaffold; baseline (speedup 1.0000x reference)
import jax
import jax.numpy as jnp
from jax.experimental import pallas as pl


def kernel(sequences, emb, W1, b1, W2, b2):
    raise NotImplementedError("write your pallas kernel here")


# speedup vs baseline: 5.1072x; 5.1072x over previous
import functools

import jax
import jax.numpy as jnp
from jax import lax
from jax.experimental import pallas as pl
from jax.experimental.pallas import tpu as pltpu
from jax.experimental.pallas import tpu_sc as plsc

_VOCAB = 100000
_SEQ = 50
_EMB = 64
_H1 = 1024
_H2 = 256
_BATCH = 4096
_IN1 = _SEQ * _EMB

_NW = 32
_N = _BATCH * _SEQ
_PER_W = _N // _NW
_CH = 128
_K = 5
_GROUP = _CH * _K
_NG = _PER_W // _GROUP


def _gather_body(seq_hbm, table_hbm, out_hbm, idx_v, rows_v, gsem):
    wid = lax.axis_index("s") * 2 + lax.axis_index("c")
    base = wid * _PER_W
    pltpu.sync_copy(seq_hbm.at[wid], idx_v)

    def group(g, carry):
        row0 = g * _GROUP
        cps = []
        for c in range(_K):
            cps.append(pltpu.async_copy(
                table_hbm.at[idx_v.at[pl.ds(row0 + c * _CH, _CH)]],
                rows_v.at[pl.ds(c * _CH, _CH)],
                gsem,
            ))
        for cp in cps:
            cp.wait()
        pltpu.sync_copy(rows_v, out_hbm.at[pl.ds(base + row0, _GROUP)])
        return carry

    lax.fori_loop(0, _NG, group, 0)


@functools.partial(
    pl.kernel,
    out_type=jax.ShapeDtypeStruct((_N, _EMB), jnp.float32),
    mesh=plsc.VectorSubcoreMesh(core_axis_name="c", subcore_axis_name="s"),
    scratch_types=[
        pltpu.VMEM((_PER_W,), jnp.int32),
        pltpu.VMEM((_GROUP, _EMB), jnp.float32),
        pltpu.SemaphoreType.DMA,
    ],
    compiler_params=pltpu.CompilerParams(use_tc_tiling_on_sc=False),
)
def _sc_gather(seq_hbm, table_hbm, out_hbm, idx_v, rows_v, gsem):
    _gather_body(seq_hbm, table_hbm, out_hbm, idx_v, rows_v, gsem)


_BB = 256


def _mlp_body(x_ref, w1_ref, b1_ref, w2_ref, b2_ref, o_ref):
    h = jnp.dot(x_ref[...], w1_ref[...], preferred_element_type=jnp.float32)
    h = h + b1_ref[...]
    o = jnp.dot(h, w2_ref[...], preferred_element_type=jnp.float32)
    o = o + b2_ref[...]
    o_ref[...] = jax.nn.sigmoid(o)


def _mlp(x, W1, b1, W2, b2):
    return pl.pallas_call(
        _mlp_body,
        grid=(_BATCH // _BB,),
        in_specs=[
            pl.BlockSpec((_BB, _IN1), lambda i: (i, 0)),
            pl.BlockSpec((_IN1, _H1), lambda i: (0, 0)),
            pl.BlockSpec((1, _H1), lambda i: (0, 0)),
            pl.BlockSpec((_H1, _H2), lambda i: (0, 0)),
            pl.BlockSpec((1, _H2), lambda i: (0, 0)),
        ],
        out_specs=pl.BlockSpec((_BB, _H2), lambda i: (i, 0)),
        out_shape=jax.ShapeDtypeStruct((_BATCH, _H2), jnp.float32),
    )(x, W1, b1.reshape(1, _H1), W2, b2.reshape(1, _H2))


def kernel(sequences, emb, W1, b1, W2, b2):
    seq = sequences.astype(jnp.int32).reshape(_NW, _PER_W)
    gathered = _sc_gather(seq, emb)
    x = gathered.reshape(_BATCH, _IN1)
    return _mlp(x, W1, b1, W2, b2)
